# Initial kernel scaffold; baseline (speedup 1.0000x reference)
import functools

import jax
import jax.numpy as jnp
from jax import lax
from jax.experimental import pallas as pl
from jax.experimental.pallas import tpu as pltpu
from jax.experimental.pallas import tpu_sc as plsc

B = 4096
L = 50
D = 64
NC, NS = 2, 16
NW = NC * NS
N = B * L
PER_W = N // NW
CHUNK = 128
NCHUNK = PER_W // CHUNK


def _body(pos_idx_hbm, rel_idx_hbm, pos_tab_hbm, rel_tab_hbm,
          pos_out_hbm, rel_out_hbm,
          idx_v, rows_v, sem):
    wid = lax.axis_index("s") * NC + lax.axis_index("c")
    base = wid * PER_W

    def run_table(idx_hbm, tab_hbm, out_hbm):
        pltpu.sync_copy(idx_hbm.at[wid], idx_v)

        def step(j, carry):
            pltpu.async_copy(tab_hbm.at[idx_v.at[j]], rows_v, sem).wait()
            pltpu.sync_copy(rows_v, out_hbm.at[pl.ds(base + j * CHUNK, CHUNK)])
            return carry

        lax.fori_loop(0, NCHUNK, step, 0)

    run_table(pos_idx_hbm, pos_tab_hbm, pos_out_hbm)
    run_table(rel_idx_hbm, rel_tab_hbm, rel_out_hbm)


@jax.jit
def _tree_embedding(position_idx, rel_idx, position_table, relation_table):
    pos_idx = position_idx.reshape(NW, NCHUNK, CHUNK).astype(jnp.int32)
    ridx = rel_idx.reshape(NW, NCHUNK, CHUNK).astype(jnp.int32)

    mesh = plsc.VectorSubcoreMesh(core_axis_name="c", subcore_axis_name="s")
    kern = pl.kernel(
        _body,
        out_type=(
            jax.ShapeDtypeStruct((N, D), jnp.float32),
            jax.ShapeDtypeStruct((N, D), jnp.float32),
        ),
        mesh=mesh,
        scratch_types=[
            pltpu.VMEM((NCHUNK, CHUNK), jnp.int32),
            pltpu.VMEM((CHUNK, D), jnp.float32),
            pltpu.SemaphoreType.DMA,
        ],
    )
    pos_out, rel_out = kern(pos_idx, ridx, position_table, relation_table)
    return (rel_out.reshape(B, L, D), pos_out.reshape(B, L, D))


def kernel(position_idx, rel_idx, position_table, relation_table):
    return _tree_embedding(position_idx, rel_idx, position_table,
                           relation_table)


# speedup vs baseline: 4.4765x; 4.4765x over previous
import functools

import jax
import jax.numpy as jnp
from jax import lax
from jax.experimental import pallas as pl
from jax.experimental.pallas import tpu as pltpu
from jax.experimental.pallas import tpu_sc as plsc

B = 4096
L = 50
D = 64
NC, NS = 2, 16
NW = NC * NS
N = B * L
PER_W = N // NW
CHUNK = 128
NCHUNK = PER_W // CHUNK


def _body(pos_idx_hbm, rel_idx_hbm, pos_tab_hbm, rel_tab_hbm,
          pos_out_hbm, rel_out_hbm,
          idx_v, rows_v, sem):
    wid = lax.axis_index("s") * NC + lax.axis_index("c")
    base = wid * PER_W

    def run_table(idx_hbm, tab_hbm, out_hbm):
        pltpu.sync_copy(idx_hbm.at[wid], idx_v)

        def step(j, carry):
            pltpu.async_copy(tab_hbm.at[idx_v.at[j]], rows_v, sem).wait()
            pltpu.sync_copy(rows_v, out_hbm.at[pl.ds(base + j * CHUNK, CHUNK)])
            return carry

        lax.fori_loop(0, NCHUNK, step, 0)

    run_table(pos_idx_hbm, pos_tab_hbm, pos_out_hbm)
    run_table(rel_idx_hbm, rel_tab_hbm, rel_out_hbm)


@jax.jit
def _tree_embedding(position_idx, rel_idx, position_table, relation_table):
    pos_idx = position_idx.reshape(NW, NCHUNK, CHUNK).astype(jnp.int32)
    ridx = rel_idx.reshape(NW, NCHUNK, CHUNK).astype(jnp.int32)

    mesh = plsc.VectorSubcoreMesh(core_axis_name="c", subcore_axis_name="s")
    kern = pl.kernel(
        _body,
        out_type=(
            jax.ShapeDtypeStruct((N, D), jnp.float32),
            jax.ShapeDtypeStruct((N, D), jnp.float32),
        ),
        mesh=mesh,
        scratch_types=[
            pltpu.VMEM((NCHUNK, CHUNK), jnp.int32),
            pltpu.VMEM((CHUNK, D), jnp.float32),
            pltpu.SemaphoreType.DMA,
        ],
        compiler_params=pltpu.CompilerParams(use_tc_tiling_on_sc=False),
    )
    pos_out, rel_out = kern(pos_idx, ridx, position_table, relation_table)
    return (rel_out.reshape(B, L, D), pos_out.reshape(B, L, D))


def kernel(position_idx, rel_idx, position_table, relation_table):
    return _tree_embedding(position_idx, rel_idx, position_table,
                           relation_table)


# speedup vs baseline: 4.9147x; 1.0979x over previous
import functools

import jax
import jax.numpy as jnp
from jax import lax
from jax.experimental import pallas as pl
from jax.experimental.pallas import tpu as pltpu
from jax.experimental.pallas import tpu_sc as plsc

B = 4096
L = 50
D = 64
NC, NS = 2, 16
NW = NC * NS
N = B * L
PER_W = N // NW
CHUNK = 128
NCHUNK = PER_W // CHUNK
R = 5
NI2 = NCHUNK // (2 * R)


def _run_table(idx_hbm, tab_hbm, out_hbm, wid, base,
               idx_v, buf_a, buf_b, gsem_a, gsem_b, wsem_a, wsem_b):
    pltpu.sync_copy(idx_hbm.at[wid], idx_v)

    def fire_gathers(rnd, buf, sem):
        for r in range(R):
            pltpu.async_copy(tab_hbm.at[idx_v.at[rnd * R + r]],
                             buf.at[pl.ds(r * CHUNK, CHUNK)], sem)

    def drain_gathers(rnd, buf, sem):
        for r in range(R):
            pltpu.make_async_copy(tab_hbm.at[idx_v.at[rnd * R + r]],
                                  buf.at[pl.ds(r * CHUNK, CHUNK)], sem).wait()

    def fire_write(rnd, buf, sem):
        pltpu.async_copy(
            buf, out_hbm.at[pl.ds(base + rnd * R * CHUNK, R * CHUNK)], sem)

    def drain_write(rnd, buf, sem):
        pltpu.make_async_copy(
            buf, out_hbm.at[pl.ds(base + rnd * R * CHUNK, R * CHUNK)],
            sem).wait()

    fire_gathers(0, buf_a, gsem_a)

    def body(i, carry):
        rnd_a = 2 * i
        rnd_b = rnd_a + 1

        @pl.when(i > 0)
        def _():
            drain_write(rnd_b - 2, buf_b, wsem_b)

        fire_gathers(rnd_b, buf_b, gsem_b)
        drain_gathers(rnd_a, buf_a, gsem_a)
        fire_write(rnd_a, buf_a, wsem_a)

        @pl.when(i < NI2 - 1)
        def _():
            drain_write(rnd_a, buf_a, wsem_a)
            fire_gathers(rnd_a + 2, buf_a, gsem_a)

        drain_gathers(rnd_b, buf_b, gsem_b)
        fire_write(rnd_b, buf_b, wsem_b)
        return carry

    lax.fori_loop(0, NI2, body, 0)
    drain_write(2 * NI2 - 2, buf_a, wsem_a)
    drain_write(2 * NI2 - 1, buf_b, wsem_b)


def _body(pos_idx_hbm, rel_idx_hbm, pos_tab_hbm, rel_tab_hbm,
          pos_out_hbm, rel_out_hbm,
          idx_v, buf_a, buf_b, gsem_a, gsem_b, wsem_a, wsem_b):
    wid = lax.axis_index("s") * NC + lax.axis_index("c")
    base = wid * PER_W
    scratch = (idx_v, buf_a, buf_b, gsem_a, gsem_b, wsem_a, wsem_b)
    _run_table(pos_idx_hbm, pos_tab_hbm, pos_out_hbm, wid, base, *scratch)
    _run_table(rel_idx_hbm, rel_tab_hbm, rel_out_hbm, wid, base, *scratch)


@jax.jit
def _tree_embedding(position_idx, rel_idx, position_table, relation_table):
    pos_idx = position_idx.reshape(NW, NCHUNK, CHUNK).astype(jnp.int32)
    ridx = rel_idx.reshape(NW, NCHUNK, CHUNK).astype(jnp.int32)

    mesh = plsc.VectorSubcoreMesh(core_axis_name="c", subcore_axis_name="s")
    kern = pl.kernel(
        _body,
        out_type=(
            jax.ShapeDtypeStruct((N, D), jnp.float32),
            jax.ShapeDtypeStruct((N, D), jnp.float32),
        ),
        mesh=mesh,
        scratch_types=[
            pltpu.VMEM((NCHUNK, CHUNK), jnp.int32),
            pltpu.VMEM((R * CHUNK, D), jnp.float32),
            pltpu.VMEM((R * CHUNK, D), jnp.float32),
            pltpu.SemaphoreType.DMA,
            pltpu.SemaphoreType.DMA,
            pltpu.SemaphoreType.DMA,
            pltpu.SemaphoreType.DMA,
        ],
        compiler_params=pltpu.CompilerParams(use_tc_tiling_on_sc=False),
    )
    pos_out, rel_out = kern(pos_idx, ridx, position_table, relation_table)
    return (rel_out.reshape(B, L, D), pos_out.reshape(B, L, D))


def kernel(position_idx, rel_idx, position_table, relation_table):
    return _tree_embedding(position_idx, rel_idx, position_table,
                           relation_table)
